# Initial kernel scaffold; baseline (speedup 1.0000x reference)
import jax
import jax.numpy as jnp
from jax.experimental import pallas as pl


def kernel(action, W_power, W_turn, W_shoot):
    raise NotImplementedError("write your pallas kernel here")


# speedup vs baseline: 1.9768x; 1.9768x over previous
import functools

import jax
import jax.numpy as jnp
from jax import lax
from jax.experimental import pallas as pl
from jax.experimental.pallas import tpu as pltpu
from jax.experimental.pallas import tpu_sc as plsc

L = 16
NC = 2
NS = 16
NW = NC * NS
CH = 128
D = 128


def _make_sc_lookup(B: int):
    per_w = B // NW
    n_chunks = per_w // CH
    mesh = plsc.VectorSubcoreMesh(core_axis_name="c", subcore_axis_name="s")

    @functools.partial(
        pl.kernel,
        mesh=mesh,
        out_type=jax.ShapeDtypeStruct((B, D), jnp.float32),
        scratch_types=[
            pltpu.VMEM((CH,), jnp.int32),
            pltpu.VMEM((CH,), jnp.int32),
            pltpu.VMEM((CH,), jnp.int32),
            pltpu.VMEM((CH,), jnp.int32),
            pltpu.VMEM((CH, D), jnp.float32),
            pltpu.SemaphoreType.DMA,
        ],
    )
    def lookup(table_hbm, p_hbm, t_hbm, s_hbm, out_hbm,
               p_v, t_v, s_v, idx_v, rows_v, sem):
        wid = lax.axis_index("s") * NC + lax.axis_index("c")
        w_base = wid * per_w

        def chunk_body(g, _):
            base = w_base + g * CH
            pltpu.sync_copy(p_hbm.at[pl.ds(base, CH)], p_v)
            pltpu.sync_copy(t_hbm.at[pl.ds(base, CH)], t_v)
            pltpu.sync_copy(s_hbm.at[pl.ds(base, CH)], s_v)

            def vec_body(i, _):
                sl = pl.ds(i * L, L)
                pi = jnp.clip(p_v[sl], 0, 2)
                ti = jnp.clip(t_v[sl], 0, 6)
                si = jnp.clip(s_v[sl], 0, 1)
                idx_v[sl] = pi * 14 + ti * 2 + si
                return 0

            lax.fori_loop(0, CH // L, vec_body, 0)
            pltpu.async_copy(table_hbm.at[idx_v], rows_v, sem).wait()
            pltpu.sync_copy(rows_v, out_hbm.at[pl.ds(base, CH)])
            return 0

        lax.fori_loop(0, n_chunks, chunk_body, 0)

    return lookup


def kernel(action, W_power, W_turn, W_shoot):
    Bdim, Tdim, _ = action.shape
    B = Bdim * Tdim

    tp = jnp.broadcast_to(W_power[:, None, None, :], (3, 7, 2, 32))
    tt = jnp.broadcast_to(W_turn[None, :, None, :], (3, 7, 2, 64))
    ts = jnp.broadcast_to(W_shoot[None, None, :, :], (3, 7, 2, 32))
    table = jnp.concatenate([tp, tt, ts], axis=-1).reshape(42, D)

    a32 = action.reshape(B, 3).astype(jnp.int32)
    p = a32[:, 0]
    t = a32[:, 1]
    s = a32[:, 2]

    out = _make_sc_lookup(B)(table, p, t, s)
    return out.reshape(Bdim, Tdim, D)


# speedup vs baseline: 35.4527x; 17.9345x over previous
import functools

import jax
import jax.numpy as jnp
from jax import lax
from jax.experimental import pallas as pl
from jax.experimental.pallas import tpu as pltpu
from jax.experimental.pallas import tpu_sc as plsc

L = 16
NC = 2
NS = 16
NW = NC * NS
CH = 128
D = 128
NROWS = 42


def _make_sc_lookup(B: int):
    per_w = B // NW
    n_chunks = per_w // CH
    mesh = plsc.VectorSubcoreMesh(core_axis_name="c", subcore_axis_name="s")

    @functools.partial(
        pl.kernel,
        mesh=mesh,
        out_type=jax.ShapeDtypeStruct((B, D), jnp.float32),
        scratch_types=[
            pltpu.VMEM_SHARED((NROWS, D), jnp.float32),
            pltpu.VMEM((2, CH), jnp.int32),
            pltpu.VMEM((2, CH), jnp.int32),
            pltpu.VMEM((2, CH), jnp.int32),
            pltpu.VMEM((2, CH), jnp.int32),
            pltpu.VMEM((2, CH, D), jnp.float32),
            pltpu.SemaphoreType.DMA,
            pltpu.SemaphoreType.DMA,
            pltpu.SemaphoreType.DMA,
            pltpu.SemaphoreType.DMA,
        ],
    )
    def lookup(table_hbm, p_hbm, t_hbm, s_hbm, out_hbm,
               table_sh, p_v, t_v, s_v, idx_v, rows_v,
               sem_a, sem_g, sem_s0, sem_s1):
        cid = lax.axis_index("c")
        sid = lax.axis_index("s")
        wid = sid * NC + cid
        w_base = wid * per_w
        sem_s = (sem_s0, sem_s1)

        @pl.when(sid == 0)
        def _stage():
            pltpu.sync_copy(table_hbm, table_sh)

        plsc.subcore_barrier()

        def load_a(g, b):
            base = w_base + g * CH
            pltpu.async_copy(p_hbm.at[pl.ds(base, CH)], p_v.at[b], sem_a)
            pltpu.async_copy(t_hbm.at[pl.ds(base, CH)], t_v.at[b], sem_a)
            pltpu.async_copy(s_hbm.at[pl.ds(base, CH)], s_v.at[b], sem_a)

        def wait_a(b):
            for ref in (p_v, t_v, s_v):
                pltpu.make_async_copy(
                    p_hbm.at[pl.ds(0, CH)], ref.at[b], sem_a).wait()

        load_a(0, 0)

        def outer(go, _):
            for b in range(2):
                g = go * 2 + b
                wait_a(b)

                @pl.when(g + 1 < n_chunks)
                def _prefetch():
                    load_a(g + 1, 1 - b)

                pb, tb, sb, ib = p_v.at[b], t_v.at[b], s_v.at[b], idx_v.at[b]

                def vec_body(i, _):
                    sl = pl.ds(i * L, L)
                    pi = jnp.clip(pb[sl], 0, 2)
                    ti = jnp.clip(tb[sl], 0, 6)
                    si = jnp.clip(sb[sl], 0, 1)
                    ib[sl] = pi * 14 + ti * 2 + si
                    return 0

                lax.fori_loop(0, CH // L, vec_body, 0)

                @pl.when(g >= 2)
                def _reuse():
                    pltpu.make_async_copy(
                        rows_v.at[b], out_hbm.at[pl.ds(0, CH)], sem_s[b]).wait()

                pltpu.async_copy(table_sh.at[ib], rows_v.at[b], sem_g).wait()
                pltpu.async_copy(
                    rows_v.at[b], out_hbm.at[pl.ds(w_base + g * CH, CH)],
                    sem_s[b])
            return 0

        lax.fori_loop(0, n_chunks // 2, outer, 0)

        pltpu.make_async_copy(
            rows_v.at[0], out_hbm.at[pl.ds(0, CH)], sem_s0).wait()
        pltpu.make_async_copy(
            rows_v.at[1], out_hbm.at[pl.ds(0, CH)], sem_s1).wait()

    return lookup


def kernel(action, W_power, W_turn, W_shoot):
    Bdim, Tdim, _ = action.shape
    B = Bdim * Tdim

    tp = jnp.broadcast_to(W_power[:, None, None, :], (3, 7, 2, 32))
    tt = jnp.broadcast_to(W_turn[None, :, None, :], (3, 7, 2, 64))
    ts = jnp.broadcast_to(W_shoot[None, None, :, :], (3, 7, 2, 32))
    table = jnp.concatenate([tp, tt, ts], axis=-1).reshape(NROWS, D)

    a32 = action.reshape(B, 3).astype(jnp.int32)
    p = a32[:, 0]
    t = a32[:, 1]
    s = a32[:, 2]

    out = _make_sc_lookup(B)(table, p, t, s)
    return out.reshape(Bdim, Tdim, D)


# speedup vs baseline: 36.3982x; 1.0267x over previous
import functools

import jax
import jax.numpy as jnp
from jax import lax
from jax.experimental import pallas as pl
from jax.experimental.pallas import tpu as pltpu
from jax.experimental.pallas import tpu_sc as plsc

L = 16
NC = 2
NS = 16
NW = NC * NS
CH = 128
D = 128
NROWS = 42


def _make_sc_lookup(B: int):
    per_w = B // NW
    n_chunks = per_w // CH
    mesh = plsc.VectorSubcoreMesh(core_axis_name="c", subcore_axis_name="s")

    @functools.partial(
        pl.kernel,
        mesh=mesh,
        out_type=jax.ShapeDtypeStruct((B, D), jnp.float32),
        scratch_types=[
            pltpu.VMEM_SHARED((NROWS, D), jnp.float32),
            pltpu.VMEM((2, CH), jnp.int32),
            pltpu.VMEM((2, CH), jnp.int32),
            pltpu.VMEM((2, CH), jnp.int32),
            pltpu.VMEM((2, CH), jnp.int32),
            pltpu.VMEM((2, CH, D), jnp.float32),
            pltpu.SemaphoreType.DMA,
            pltpu.SemaphoreType.DMA,
            pltpu.SemaphoreType.DMA,
            pltpu.SemaphoreType.DMA,
            pltpu.SemaphoreType.DMA,
        ],
    )
    def lookup(table_hbm, p_hbm, t_hbm, s_hbm, out_hbm,
               table_sh, p_v, t_v, s_v, idx_v, rows_v,
               sem_a, sem_g0, sem_g1, sem_s0, sem_s1):
        cid = lax.axis_index("c")
        sid = lax.axis_index("s")
        wid = sid * NC + cid
        w_base = wid * per_w
        sem_g = (sem_g0, sem_g1)
        sem_s = (sem_s0, sem_s1)

        @pl.when(sid == 0)
        def _stage():
            pltpu.sync_copy(table_hbm, table_sh)

        plsc.subcore_barrier()

        def load_a(g, b):
            base = w_base + g * CH
            pltpu.async_copy(p_hbm.at[pl.ds(base, CH)], p_v.at[b], sem_a)
            pltpu.async_copy(t_hbm.at[pl.ds(base, CH)], t_v.at[b], sem_a)
            pltpu.async_copy(s_hbm.at[pl.ds(base, CH)], s_v.at[b], sem_a)

        def wait_a(b):
            for ref in (p_v, t_v, s_v):
                pltpu.make_async_copy(
                    p_hbm.at[pl.ds(0, CH)], ref.at[b], sem_a).wait()

        load_a(0, 0)

        def outer(go, _):
            for b in range(2):
                g = go * 2 + b
                wait_a(b)

                @pl.when(g + 1 < n_chunks)
                def _prefetch():
                    load_a(g + 1, 1 - b)

                pb, tb, sb, ib = p_v.at[b], t_v.at[b], s_v.at[b], idx_v.at[b]

                def vec_body(i, _):
                    sl = pl.ds(i * L, L)
                    pi = jnp.clip(pb[sl], 0, 2)
                    ti = jnp.clip(tb[sl], 0, 6)
                    si = jnp.clip(sb[sl], 0, 1)
                    ib[sl] = pi * 14 + ti * 2 + si
                    return 0

                lax.fori_loop(0, CH // L, vec_body, 0)

                @pl.when(g >= 2)
                def _reuse():
                    pltpu.make_async_copy(
                        rows_v.at[b], out_hbm.at[pl.ds(0, CH)], sem_s[b]).wait()

                pltpu.async_copy(table_sh.at[ib], rows_v.at[b], sem_g[b])

                @pl.when(g >= 1)
                def _store_prev():
                    pltpu.make_async_copy(
                        table_sh.at[idx_v.at[1 - b]], rows_v.at[1 - b],
                        sem_g[1 - b]).wait()
                    pltpu.async_copy(
                        rows_v.at[1 - b],
                        out_hbm.at[pl.ds(w_base + (g - 1) * CH, CH)],
                        sem_s[1 - b])
            return 0

        lax.fori_loop(0, n_chunks // 2, outer, 0)

        b_last = (n_chunks - 1) % 2
        pltpu.make_async_copy(
            table_sh.at[idx_v.at[b_last]], rows_v.at[b_last],
            sem_g[b_last]).wait()
        pltpu.async_copy(
            rows_v.at[b_last],
            out_hbm.at[pl.ds(w_base + (n_chunks - 1) * CH, CH)],
            sem_s[b_last])
        pltpu.make_async_copy(
            rows_v.at[0], out_hbm.at[pl.ds(0, CH)], sem_s0).wait()
        pltpu.make_async_copy(
            rows_v.at[1], out_hbm.at[pl.ds(0, CH)], sem_s1).wait()

    return lookup


def kernel(action, W_power, W_turn, W_shoot):
    Bdim, Tdim, _ = action.shape
    B = Bdim * Tdim

    tp = jnp.broadcast_to(W_power[:, None, None, :], (3, 7, 2, 32))
    tt = jnp.broadcast_to(W_turn[None, :, None, :], (3, 7, 2, 64))
    ts = jnp.broadcast_to(W_shoot[None, None, :, :], (3, 7, 2, 32))
    table = jnp.concatenate([tp, tt, ts], axis=-1).reshape(NROWS, D)

    a32 = action.reshape(B, 3).astype(jnp.int32)
    p = a32[:, 0]
    t = a32[:, 1]
    s = a32[:, 2]

    out = _make_sc_lookup(B)(table, p, t, s)
    return out.reshape(Bdim, Tdim, D)


# speedup vs baseline: 38.4583x; 1.0566x over previous
import functools

import jax
import jax.numpy as jnp
from jax import lax
from jax.experimental import pallas as pl
from jax.experimental.pallas import tpu as pltpu
from jax.experimental.pallas import tpu_sc as plsc

L = 16
NC = 2
NS = 16
NW = NC * NS
G = 128
NG = 2
CH = G * NG
D = 128
NROWS = 42


def _make_sc_lookup(B: int):
    per_w = B // NW
    n_chunks = per_w // CH
    mesh = plsc.VectorSubcoreMesh(core_axis_name="c", subcore_axis_name="s")

    @functools.partial(
        pl.kernel,
        mesh=mesh,
        out_type=jax.ShapeDtypeStruct((B, D), jnp.float32),
        scratch_types=[
            pltpu.VMEM_SHARED((NROWS, D), jnp.float32),
            pltpu.VMEM((2, CH), jnp.int32),
            pltpu.VMEM((2, CH), jnp.int32),
            pltpu.VMEM((2, CH), jnp.int32),
            pltpu.VMEM((2, NG, G), jnp.int32),
            pltpu.VMEM((2, CH, D), jnp.float32),
            pltpu.SemaphoreType.DMA,
            pltpu.SemaphoreType.DMA,
            pltpu.SemaphoreType.DMA,
            pltpu.SemaphoreType.DMA,
            pltpu.SemaphoreType.DMA,
        ],
    )
    def lookup(table_hbm, p_hbm, t_hbm, s_hbm, out_hbm,
               table_sh, p_v, t_v, s_v, idx_v, rows_v,
               sem_a, sem_g0, sem_g1, sem_s0, sem_s1):
        cid = lax.axis_index("c")
        sid = lax.axis_index("s")
        wid = sid * NC + cid
        w_base = wid * per_w
        sem_g = (sem_g0, sem_g1)
        sem_s = (sem_s0, sem_s1)

        @pl.when(sid == 0)
        def _stage():
            pltpu.sync_copy(table_hbm, table_sh)

        plsc.subcore_barrier()

        def load_a(g, b):
            base = w_base + g * CH
            pltpu.async_copy(p_hbm.at[pl.ds(base, CH)], p_v.at[b], sem_a)
            pltpu.async_copy(t_hbm.at[pl.ds(base, CH)], t_v.at[b], sem_a)
            pltpu.async_copy(s_hbm.at[pl.ds(base, CH)], s_v.at[b], sem_a)

        def wait_a(b):
            for ref in (p_v, t_v, s_v):
                pltpu.make_async_copy(
                    p_hbm.at[pl.ds(0, CH)], ref.at[b], sem_a).wait()

        def gather(b):
            for j in range(NG):
                pltpu.async_copy(
                    table_sh.at[idx_v.at[b, j]],
                    rows_v.at[b, pl.ds(j * G, G)], sem_g[b])

        def wait_gather(b):
            for j in range(NG):
                pltpu.make_async_copy(
                    table_sh.at[idx_v.at[b, j]],
                    rows_v.at[b, pl.ds(j * G, G)], sem_g[b]).wait()

        load_a(0, 0)

        def outer(go, _):
            for b in range(2):
                g = go * 2 + b
                wait_a(b)

                @pl.when(g + 1 < n_chunks)
                def _prefetch():
                    load_a(g + 1, 1 - b)

                pb, tb, sb = p_v.at[b], t_v.at[b], s_v.at[b]

                for j in range(NG):
                    ib = idx_v.at[b, j]

                    def vec_body(i, _):
                        src = pl.ds(j * G + i * L, L)
                        dst = pl.ds(i * L, L)
                        pi = jnp.clip(pb[src], 0, 2)
                        ti = jnp.clip(tb[src], 0, 6)
                        si = jnp.clip(sb[src], 0, 1)
                        ib[dst] = pi * 14 + ti * 2 + si
                        return 0

                    lax.fori_loop(0, G // L, vec_body, 0)

                @pl.when(g >= 2)
                def _reuse():
                    pltpu.make_async_copy(
                        rows_v.at[b], out_hbm.at[pl.ds(0, CH)], sem_s[b]).wait()

                gather(b)

                @pl.when(g >= 1)
                def _store_prev():
                    wait_gather(1 - b)
                    pltpu.async_copy(
                        rows_v.at[1 - b],
                        out_hbm.at[pl.ds(w_base + (g - 1) * CH, CH)],
                        sem_s[1 - b])
            return 0

        lax.fori_loop(0, n_chunks // 2, outer, 0)

        b_last = (n_chunks - 1) % 2
        wait_gather(b_last)
        pltpu.async_copy(
            rows_v.at[b_last],
            out_hbm.at[pl.ds(w_base + (n_chunks - 1) * CH, CH)],
            sem_s[b_last])
        pltpu.make_async_copy(
            rows_v.at[0], out_hbm.at[pl.ds(0, CH)], sem_s0).wait()
        pltpu.make_async_copy(
            rows_v.at[1], out_hbm.at[pl.ds(0, CH)], sem_s1).wait()

    return lookup


def kernel(action, W_power, W_turn, W_shoot):
    Bdim, Tdim, _ = action.shape
    B = Bdim * Tdim

    tp = jnp.broadcast_to(W_power[:, None, None, :], (3, 7, 2, 32))
    tt = jnp.broadcast_to(W_turn[None, :, None, :], (3, 7, 2, 64))
    ts = jnp.broadcast_to(W_shoot[None, None, :, :], (3, 7, 2, 32))
    table = jnp.concatenate([tp, tt, ts], axis=-1).reshape(NROWS, D)

    a32 = action.reshape(B, 3).astype(jnp.int32)
    p = a32[:, 0]
    t = a32[:, 1]
    s = a32[:, 2]

    out = _make_sc_lookup(B)(table, p, t, s)
    return out.reshape(Bdim, Tdim, D)
